# baseline (speedup 1.0000x reference)
import functools

import jax
import jax.numpy as jnp
from jax import lax
from jax.experimental import pallas as pl
from jax.experimental.pallas import tpu as pltpu
from jax.experimental.pallas import tpu_sc as plsc

B = 4096
T = 200
D = 64
N = B * T

NC = 2
NS = 16
NW = NC * NS

ROWS_PER_W = B // NW
CHUNK = T
NCHUNK = ROWS_PER_W
NBUF = 4
NSUPER = NCHUNK // NBUF


def _make_kernel():
    mesh = plsc.VectorSubcoreMesh(core_axis_name="c", subcore_axis_name="s")

    scratch = (
        [pltpu.VMEM((CHUNK,), jnp.int32) for _ in range(NBUF)]
        + [pltpu.VMEM((CHUNK,), jnp.int32) for _ in range(NBUF)]
        + [pltpu.VMEM((CHUNK, D), jnp.float32) for _ in range(NBUF)]
        + [pltpu.VMEM((CHUNK, D), jnp.float32) for _ in range(NBUF)]
        + [pltpu.VMEM((T, D), jnp.float32)]
        + [
            pltpu.VMEM_SHARED((1000, D), jnp.float32),
            pltpu.VMEM_SHARED((201, D), jnp.float32),
        ]
        + [pltpu.SemaphoreType.DMA for _ in range(3 * NBUF)]
    )

    @functools.partial(
        pl.kernel,
        out_type=jax.ShapeDtypeStruct((N, D), jnp.float32),
        mesh=mesh,
        scratch_types=scratch,
        compiler_params=pltpu.CompilerParams(use_tc_tiling_on_sc=False),
    )
    def embed_kernel(xt_hbm, yt_hbm, xtab_hbm, ytab_hbm, ptab_hbm, out_hbm,
                     *refs):
        xidx = refs[0:NBUF]
        yidx = refs[NBUF:2 * NBUF]
        bufX = refs[2 * NBUF:3 * NBUF]
        bufY = refs[3 * NBUF:4 * NBUF]
        tmpl = refs[4 * NBUF]
        xtab_sp, ytab_sp = refs[4 * NBUF + 1:4 * NBUF + 3]
        sems = refs[4 * NBUF + 3:]
        semA = sems[0:NBUF]
        semB = sems[NBUF:2 * NBUF]
        semD = sems[2 * NBUF:3 * NBUF]

        wid = lax.axis_index("s") * NC + lax.axis_index("c")
        base_tok = wid * (ROWS_PER_W * T)

        @pl.when(lax.axis_index("s") == 0)
        def _():
            pltpu.sync_copy(xtab_hbm, xtab_sp)
            pltpu.sync_copy(ytab_hbm, ytab_sp)

        pltpu.sync_copy(ptab_hbm, tmpl)
        plsc.subcore_barrier()

        def superstep(g, _):
            descB = []
            for b in range(NBUF):
                ci = g * NBUF + b
                tok0 = base_tok + ci * CHUNK

                @pl.when(g > 0)
                def _(b=b):
                    pltpu.make_async_copy(
                        out_hbm.at[pl.ds(0, CHUNK)], bufX[b], semD[b]
                    ).wait()
                    pltpu.make_async_copy(
                        xt_hbm.at[pl.ds(0, CHUNK)], xidx[b], semA[b]
                    ).wait()
                    pltpu.make_async_copy(
                        yt_hbm.at[pl.ds(0, CHUNK)], yidx[b], semA[b]
                    ).wait()

                @pl.when(g == 0)
                def _(b=b, tok0=tok0):
                    pltpu.sync_copy(xt_hbm.at[pl.ds(tok0, CHUNK)], xidx[b])
                    pltpu.sync_copy(yt_hbm.at[pl.ds(tok0, CHUNK)], yidx[b])

                descB.append((
                    pltpu.async_copy(xtab_sp.at[xidx[b]], bufX[b], semB[b]),
                    pltpu.async_copy(ytab_sp.at[yidx[b]], bufY[b], semB[b]),
                ))

            for b in range(NBUF):
                ci = g * NBUF + b
                tok0 = base_tok + ci * CHUNK
                descB[b][0].wait()
                descB[b][1].wait()

                def vrow(j, _, b=b):
                    for r in range(2):
                        row = j * 2 + r
                        for k in range(D // 16):
                            sl = pl.ds(k * 16, 16)
                            bufX[b][row, sl] = (
                                bufX[b][row, sl]
                                + bufY[b][row, sl]
                                + tmpl[row, sl]
                            )
                    return ()

                lax.fori_loop(0, CHUNK // 2, vrow, ())

                pltpu.async_copy(bufX[b], out_hbm.at[pl.ds(tok0, CHUNK)],
                                 semD[b])

                @pl.when(g + 1 < NSUPER)
                def _(b=b, tok0=tok0):
                    tok1 = tok0 + NBUF * CHUNK
                    pltpu.async_copy(
                        xt_hbm.at[pl.ds(tok1, CHUNK)], xidx[b], semA[b])
                    pltpu.async_copy(
                        yt_hbm.at[pl.ds(tok1, CHUNK)], yidx[b], semA[b])
            return ()

        lax.fori_loop(0, NSUPER, superstep, ())

        for b in range(NBUF):
            pltpu.make_async_copy(
                out_hbm.at[pl.ds(0, CHUNK)], bufX[b], semD[b]
            ).wait()

    return embed_kernel


_kernel = _make_kernel()


@jax.jit
def kernel(x_tokens, y_tokens, x_table, y_table, pos_table):
    xt = x_tokens.reshape(N).astype(jnp.int32)
    yt = y_tokens.reshape(N).astype(jnp.int32)
    out = _kernel(xt, yt, x_table, y_table, pos_table)
    return out.reshape(B, T, D)


# speedup vs baseline: 1.0458x; 1.0458x over previous
import functools

import jax
import jax.numpy as jnp
from jax import lax
from jax.experimental import pallas as pl
from jax.experimental.pallas import tpu as pltpu
from jax.experimental.pallas import tpu_sc as plsc

B = 4096
T = 200
D = 64
N = B * T

NC = 2
NS = 16
NW = NC * NS

ROWS_PER_W = B // NW
CHUNK = T
NCHUNK = ROWS_PER_W
NBUF = 4
K = 2
NSUPER = NCHUNK // NBUF


def _make_kernel():
    mesh = plsc.VectorSubcoreMesh(core_axis_name="c", subcore_axis_name="s")

    scratch = (
        [pltpu.VMEM((CHUNK,), jnp.int32) for _ in range(NBUF)]
        + [pltpu.VMEM((CHUNK,), jnp.int32) for _ in range(NBUF)]
        + [pltpu.VMEM((CHUNK, D), jnp.float32) for _ in range(NBUF)]
        + [pltpu.VMEM((CHUNK, D), jnp.float32) for _ in range(NBUF)]
        + [pltpu.VMEM((T, D), jnp.float32)]
        + [
            pltpu.VMEM_SHARED((1000, D), jnp.float32),
            pltpu.VMEM_SHARED((201, D), jnp.float32),
        ]
        + [pltpu.SemaphoreType.DMA for _ in range(3 * NBUF)]
    )

    @functools.partial(
        pl.kernel,
        out_type=jax.ShapeDtypeStruct((N, D), jnp.float32),
        mesh=mesh,
        scratch_types=scratch,
        compiler_params=pltpu.CompilerParams(use_tc_tiling_on_sc=False),
    )
    def embed_kernel(xt_hbm, yt_hbm, xtab_hbm, ytab_hbm, ptab_hbm, out_hbm,
                     *refs):
        xidx = refs[0:NBUF]
        yidx = refs[NBUF:2 * NBUF]
        bufX = refs[2 * NBUF:3 * NBUF]
        bufY = refs[3 * NBUF:4 * NBUF]
        tmpl = refs[4 * NBUF]
        xtab_sp, ytab_sp = refs[4 * NBUF + 1:4 * NBUF + 3]
        sems = refs[4 * NBUF + 3:]
        semA = sems[0:NBUF]
        semB = sems[NBUF:2 * NBUF]
        semD = sems[2 * NBUF:3 * NBUF]

        wid = lax.axis_index("s") * NC + lax.axis_index("c")
        base_tok = wid * (ROWS_PER_W * T)

        @pl.when(lax.axis_index("s") == 0)
        def _():
            pltpu.sync_copy(xtab_hbm, xtab_sp)
            pltpu.sync_copy(ytab_hbm, ytab_sp)

        pltpu.sync_copy(ptab_hbm, tmpl)
        plsc.subcore_barrier()

        def drain(sem, dst):
            pltpu.make_async_copy(out_hbm.at[pl.ds(0, CHUNK)], dst, sem).wait()

        def drain_idx(sem, dst):
            pltpu.make_async_copy(xt_hbm.at[pl.ds(0, CHUNK)], dst, sem).wait()

        def combine_store(bj, tokj):
            drain(semB[bj], bufX[bj])
            drain(semB[bj], bufY[bj])

            def vrow(j, _):
                for r in range(4):
                    row = j * 4 + r
                    for k in range(D // 16):
                        sl = pl.ds(k * 16, 16)
                        bufX[bj][row, sl] = (
                            bufX[bj][row, sl]
                            + bufY[bj][row, sl]
                            + tmpl[row, sl]
                        )
                return ()

            lax.fori_loop(0, CHUNK // 4, vrow, ())

            pltpu.async_copy(bufX[bj], out_hbm.at[pl.ds(tokj, CHUNK)],
                             semD[bj])

        def superstep(g, _):
            for b in range(NBUF):
                ci = g * NBUF + b
                tok0 = base_tok + ci * CHUNK

                @pl.when(g > 0)
                def _(b=b):
                    drain(semD[b], bufX[b])
                    drain_idx(semA[b], xidx[b])
                    drain_idx(semA[b], yidx[b])

                @pl.when(g == 0)
                def _(b=b, tok0=tok0):
                    pltpu.sync_copy(xt_hbm.at[pl.ds(tok0, CHUNK)], xidx[b])
                    pltpu.sync_copy(yt_hbm.at[pl.ds(tok0, CHUNK)], yidx[b])

                pltpu.async_copy(xtab_sp.at[xidx[b]], bufX[b], semB[b])
                pltpu.async_copy(ytab_sp.at[yidx[b]], bufY[b], semB[b])

                bj = (b - K) % NBUF
                cj = ci - K
                tokj = base_tok + cj * CHUNK

                @pl.when(cj >= 0)
                def _(bj=bj, tokj=tokj):
                    combine_store(bj, tokj)

                @pl.when(jnp.logical_and(cj >= 0, cj + NBUF < NCHUNK))
                def _(bj=bj, tokj=tokj):
                    tok1 = tokj + NBUF * CHUNK
                    pltpu.async_copy(
                        xt_hbm.at[pl.ds(tok1, CHUNK)], xidx[bj], semA[bj])
                    pltpu.async_copy(
                        yt_hbm.at[pl.ds(tok1, CHUNK)], yidx[bj], semA[bj])
            return ()

        lax.fori_loop(0, NSUPER, superstep, ())

        for bj in range(NBUF - K, NBUF):
            cj = NCHUNK - NBUF + bj
            combine_store(bj, base_tok + cj * CHUNK)
        for b in range(NBUF):
            drain(semD[b], bufX[b])

    return embed_kernel


_kernel = _make_kernel()


@jax.jit
def kernel(x_tokens, y_tokens, x_table, y_table, pos_table):
    xt = x_tokens.reshape(N).astype(jnp.int32)
    yt = y_tokens.reshape(N).astype(jnp.int32)
    out = _kernel(xt, yt, x_table, y_table, pos_table)
    return out.reshape(B, T, D)
